# baseline (speedup 1.0000x reference)
import functools

import jax
import jax.numpy as jnp
from jax import lax
from jax.experimental import pallas as pl
from jax.experimental.pallas import tpu as pltpu
from jax.experimental.pallas import tpu_sc as plsc

NC = 2
NS = 16
NW = NC * NS
LANES = 16
PACK = 4
CHUNK = 512
VBLK = 32768
SUB = VBLK // PACK
assert VBLK & (VBLK - 1) == 0, "packing bit math needs power-of-two VBLK"
LB = VBLK.bit_length() - 1
LS = SUB.bit_length() - 1


RCHUNK = 256


def _pack_kernel(t_ref, c_ref, to_ref, co_ref):
    eqs = [jnp.eye(32, 128, k=q * 32, dtype=jnp.bfloat16)
           for q in range(PACK)]
    for r in range(0, SUB, RCHUNK):
        for ref, oref in ((t_ref, to_ref), (c_ref, co_ref)):
            acc = None
            for q in range(PACK):
                xq = ref[:, q * SUB + r:q * SUB + r + RCHUNK]
                y = lax.dot_general(
                    xq.astype(jnp.bfloat16), eqs[q],
                    (((0,), (0,)), ((), ())),
                    precision=lax.Precision.DEFAULT,
                    preferred_element_type=jnp.float32)
                acc = y if acc is None else acc + y
            oref[pl.ds(r, RCHUNK), :] = acc


def _pack_tables(ttab_t, ctab_t):
    E, V = ttab_t.shape
    grid = (V + VBLK - 1) // VBLK
    rows = grid * SUB
    return pl.pallas_call(
        _pack_kernel,
        grid=(grid,),
        in_specs=[
            pl.BlockSpec((E, VBLK), lambda i: (0, i)),
            pl.BlockSpec((E, VBLK), lambda i: (0, i)),
        ],
        out_specs=[
            pl.BlockSpec((SUB, PACK * E), lambda i: (i, 0)),
            pl.BlockSpec((SUB, PACK * E), lambda i: (i, 0)),
        ],
        out_shape=[
            jax.ShapeDtypeStruct((rows, PACK * E), jnp.float32),
            jax.ShapeDtypeStruct((rows, PACK * E), jnp.float32),
        ],
    )(ttab_t, ctab_t)


def _packed_row(v):
    return ((v >> LB) << LS) | (v & (SUB - 1))


@functools.partial(jax.jit, static_argnames=("B", "C", "E"))
def _anime2vec_sc(target, ctx_flat, ttab_t, ctab_t, *, B, C, E):
    ttab_packed, ctab_packed = _pack_tables(ttab_t, ctab_t)

    BPW = B // NW
    RPW = BPW * C
    NP = 16
    NSLOT = 4
    PB = BPW // NP
    PR = RPW // NP
    GPP = PR // LANES
    mesh = plsc.VectorSubcoreMesh(core_axis_name="c", subcore_axis_name="s")

    @functools.partial(
        pl.kernel,
        mesh=mesh,
        out_type=jax.ShapeDtypeStruct((C, B), jnp.float32),
        scratch_types=[
            pltpu.VMEM((BPW,), jnp.int32),
            pltpu.VMEM((C, BPW), jnp.int32),
            pltpu.VMEM((RPW,), jnp.int32),
            pltpu.VMEM((C, BPW), jnp.float32),
            pltpu.VMEM((BPW,), jnp.int32),
            pltpu.VMEM((RPW,), jnp.int32),
            pltpu.VMEM((NSLOT, PB, PACK * E), jnp.float32),
            pltpu.VMEM((NSLOT, PR, PACK * E), jnp.float32),
            pltpu.VMEM((RPW,), jnp.float32),
            pltpu.VMEM((LANES, LANES), jnp.float32),
        ] + [pltpu.SemaphoreType.DMA] * NSLOT,
        compiler_params=pltpu.CompilerParams(needs_layout_passes=False),
    )
    def k(tgt_hbm, ctx_hbm, ttab_hbm, ctab_hbm, out_hbm,
          tgt_idx, ctx_idx2, ctx_idx, out_t, tgt_rid, ctx_rid, tbuf, cbuf,
          out_v, scr, *sems):
        wid = lax.axis_index("s") * NC + lax.axis_index("c")
        b0 = wid * BPW
        r0 = wid * RPW

        pltpu.sync_copy(tgt_hbm.at[pl.ds(b0, BPW)], tgt_idx)
        pltpu.sync_copy(ctx_hbm.at[:, pl.ds(b0, BPW)], ctx_idx2)

        iota16_ = lax.iota(jnp.int32, 16)

        for c in range(C):
            @pl.loop(0, BPW // LANES)
            def _(j, c=c):
                bv = j * LANES + iota16_
                vals = plsc.load_gather(
                    ctx_idx2, [jnp.full((LANES,), c, jnp.int32), bv])
                plsc.store_scatter(ctx_idx, [bv * C + c], vals)

        @pl.loop(0, BPW // LANES)
        def _(j):
            tgt_rid[pl.ds(j * LANES, LANES)] = _packed_row(
                tgt_idx[pl.ds(j * LANES, LANES)])

        @pl.loop(0, RPW // LANES)
        def _(j):
            ctx_rid[pl.ds(j * LANES, LANES)] = _packed_row(
                ctx_idx[pl.ds(j * LANES, LANES)])

        def fire(p):
            slot = p % NSLOT
            sem = sems[slot]
            hs = [pltpu.async_copy(
                ttab_hbm.at[tgt_rid.at[pl.ds(p * PB, PB)]],
                tbuf.at[slot], sem)]
            for j in range(0, PR, CHUNK):
                n = min(CHUNK, PR - j)
                hs.append(pltpu.async_copy(
                    ctab_hbm.at[ctx_rid.at[pl.ds(p * PR + j, n)]],
                    cbuf.at[slot, pl.ds(j, n)], sem))
            return hs

        iota16 = lax.iota(jnp.int32, 16)
        pend = {p: fire(p) for p in range(NSLOT)}

        for p in range(NP):
            slot = p % NSLOT
            for h in pend.pop(p):
                h.wait()

            @pl.loop(0, GPP)
            def _(g, p=p, slot=slot):
                rl0 = p * PR + g * LANES
                civ = ctx_idx[pl.ds(rl0, LANES)]
                c_offv = ((civ >> LS) & 3) << 5
                bvec = (rl0 + iota16) // C
                tbv = bvec - p * PB
                tiv = plsc.load_gather(tgt_idx, [bvec])
                t_offv = ((tiv >> LS) & 3) << 5
                for j in range(LANES):
                    rr = g * LANES + j
                    c_off = c_offv[j]
                    t_off = t_offv[j]
                    tb = tbv[j]
                    prod = (cbuf[slot, rr, pl.ds(c_off, LANES)]
                            * tbuf[slot, tb, pl.ds(t_off, LANES)]
                            + cbuf[slot, rr, pl.ds(c_off + LANES, LANES)]
                            * tbuf[slot, tb, pl.ds(t_off + LANES, LANES)])
                    scr[j, :] = prod
                acc = plsc.load_gather(scr, [iota16,
                                             jnp.zeros((LANES,), jnp.int32)])
                for e in range(1, LANES):
                    acc = acc + plsc.load_gather(
                        scr, [iota16, jnp.full((LANES,), e, jnp.int32)])
                out_v[pl.ds(p * PR + g * LANES, LANES)] = acc

            if p + NSLOT < NP:
                pend[p + NSLOT] = fire(p + NSLOT)

        for c in range(C):
            @pl.loop(0, BPW // LANES)
            def _(j, c=c):
                bv = j * LANES + iota16_
                vals = plsc.load_gather(out_v, [bv * C + c])
                out_t[c, pl.ds(j * LANES, LANES)] = vals

        pltpu.sync_copy(out_t, out_hbm.at[:, pl.ds(b0, BPW)])

    return k(target, ctx_flat, ttab_packed, ctab_packed)


def kernel(target, context, target_table, context_table):
    B, = target.shape
    _, C = context.shape
    _, E = target_table.shape
    out_t = _anime2vec_sc(
        target,
        context.T,
        target_table.T,
        context_table.T,
        B=B, C=C, E=E)
    return out_t.T


# speedup vs baseline: 1.0128x; 1.0128x over previous
import functools

import jax
import jax.numpy as jnp
from jax import lax
from jax.experimental import pallas as pl
from jax.experimental.pallas import tpu as pltpu
from jax.experimental.pallas import tpu_sc as plsc

NC = 2
NS = 16
NW = NC * NS
LANES = 16
PACK = 4
CHUNK = 512
VBLK = 32768
SUB = VBLK // PACK
assert VBLK & (VBLK - 1) == 0, "packing bit math needs power-of-two VBLK"
LB = VBLK.bit_length() - 1
LS = SUB.bit_length() - 1


RCHUNK = 256


def _pack_kernel(t_ref, c_ref, to_ref, co_ref):
    eqs = [jnp.eye(32, 128, k=q * 32, dtype=jnp.bfloat16)
           for q in range(PACK)]
    for r in range(0, SUB, RCHUNK):
        for ref, oref in ((t_ref, to_ref), (c_ref, co_ref)):
            acc = None
            for q in range(PACK):
                xq = ref[:, q * SUB + r:q * SUB + r + RCHUNK]
                y = lax.dot_general(
                    xq.astype(jnp.bfloat16), eqs[q],
                    (((0,), (0,)), ((), ())),
                    precision=lax.Precision.DEFAULT,
                    preferred_element_type=jnp.float32)
                acc = y if acc is None else acc + y
            oref[pl.ds(r, RCHUNK), :] = acc


def _pack_tables(ttab_t, ctab_t):
    E, V = ttab_t.shape
    grid = (V + VBLK - 1) // VBLK
    rows = grid * SUB
    return pl.pallas_call(
        _pack_kernel,
        grid=(grid,),
        in_specs=[
            pl.BlockSpec((E, VBLK), lambda i: (0, i)),
            pl.BlockSpec((E, VBLK), lambda i: (0, i)),
        ],
        out_specs=[
            pl.BlockSpec((SUB, PACK * E), lambda i: (i, 0)),
            pl.BlockSpec((SUB, PACK * E), lambda i: (i, 0)),
        ],
        out_shape=[
            jax.ShapeDtypeStruct((rows, PACK * E), jnp.float32),
            jax.ShapeDtypeStruct((rows, PACK * E), jnp.float32),
        ],
    )(ttab_t, ctab_t)


def _packed_row(v):
    return ((v >> LB) << LS) | (v & (SUB - 1))


@functools.partial(jax.jit, static_argnames=("B", "C", "E"))
def _anime2vec_sc(target, ctx_flat, ttab_t, ctab_t, *, B, C, E):
    ttab_packed, ctab_packed = _pack_tables(ttab_t, ctab_t)

    BPW = B // NW
    RPW = BPW * C
    NP = 8
    NSLOT = 2
    PB = BPW // NP
    PR = RPW // NP
    GPP = PR // LANES
    mesh = plsc.VectorSubcoreMesh(core_axis_name="c", subcore_axis_name="s")

    @functools.partial(
        pl.kernel,
        mesh=mesh,
        out_type=jax.ShapeDtypeStruct((C, B), jnp.float32),
        scratch_types=[
            pltpu.VMEM((BPW,), jnp.int32),
            pltpu.VMEM((C, BPW), jnp.int32),
            pltpu.VMEM((RPW,), jnp.int32),
            pltpu.VMEM((C, BPW), jnp.float32),
            pltpu.VMEM((BPW,), jnp.int32),
            pltpu.VMEM((RPW,), jnp.int32),
            pltpu.VMEM((NSLOT, PB, PACK * E), jnp.float32),
            pltpu.VMEM((NSLOT, PR, PACK * E), jnp.float32),
            pltpu.VMEM((RPW,), jnp.float32),
            pltpu.VMEM((LANES * C, LANES), jnp.float32),
        ] + [pltpu.SemaphoreType.DMA] * NSLOT,
        compiler_params=pltpu.CompilerParams(needs_layout_passes=False),
    )
    def k(tgt_hbm, ctx_hbm, ttab_hbm, ctab_hbm, out_hbm,
          tgt_idx, ctx_idx2, ctx_idx, out_t, tgt_rid, ctx_rid, tbuf, cbuf,
          out_v, scr, *sems):
        wid = lax.axis_index("s") * NC + lax.axis_index("c")
        b0 = wid * BPW
        r0 = wid * RPW

        pltpu.sync_copy(tgt_hbm.at[pl.ds(b0, BPW)], tgt_idx)
        pltpu.sync_copy(ctx_hbm.at[:, pl.ds(b0, BPW)], ctx_idx2)

        iota16_ = lax.iota(jnp.int32, 16)

        for c in range(C):
            @pl.loop(0, BPW // LANES)
            def _(j, c=c):
                bv = j * LANES + iota16_
                vals = plsc.load_gather(
                    ctx_idx2, [jnp.full((LANES,), c, jnp.int32), bv])
                plsc.store_scatter(ctx_idx, [bv * C + c], vals)

        @pl.loop(0, BPW // LANES)
        def _(j):
            tgt_rid[pl.ds(j * LANES, LANES)] = _packed_row(
                tgt_idx[pl.ds(j * LANES, LANES)])

        @pl.loop(0, RPW // LANES)
        def _(j):
            ctx_rid[pl.ds(j * LANES, LANES)] = _packed_row(
                ctx_idx[pl.ds(j * LANES, LANES)])

        def fire(p):
            slot = p % NSLOT
            sem = sems[slot]
            hs = [pltpu.async_copy(
                ttab_hbm.at[tgt_rid.at[pl.ds(p * PB, PB)]],
                tbuf.at[slot], sem)]
            for j in range(0, PR, CHUNK):
                n = min(CHUNK, PR - j)
                hs.append(pltpu.async_copy(
                    ctab_hbm.at[ctx_rid.at[pl.ds(p * PR + j, n)]],
                    cbuf.at[slot, pl.ds(j, n)], sem))
            return hs

        iota16 = lax.iota(jnp.int32, 16)
        pend = {p: fire(p) for p in range(NSLOT)}

        for p in range(NP):
            slot = p % NSLOT
            for h in pend.pop(p):
                h.wait()

            @pl.loop(0, PR // (LANES * C))
            def _(sg, p=p, slot=slot):
                base_r = p * PR + sg * LANES * C
                tb0 = sg * LANES
                tiv = tgt_idx[pl.ds(p * PB + tb0, LANES)]
                t_offv = ((tiv >> LS) & 3) << 5
                civs = [ctx_idx[pl.ds(base_r + v * LANES, LANES)]
                        for v in range(C)]
                c_offvs = [((cv >> LS) & 3) << 5 for cv in civs]
                for k in range(LANES):
                    t_off = t_offv[k]
                    tb = tb0 + k
                    t0 = tbuf[slot, tb, pl.ds(t_off, LANES)]
                    t1 = tbuf[slot, tb, pl.ds(t_off + LANES, LANES)]
                    for c in range(C):
                        sr = k * C + c
                        rr = sg * LANES * C + sr
                        c_off = c_offvs[sr // LANES][sr % LANES]
                        prod = (cbuf[slot, rr, pl.ds(c_off, LANES)] * t0
                                + cbuf[slot, rr, pl.ds(c_off + LANES,
                                                       LANES)] * t1)
                        scr[sr, :] = prod
                for blk in range(C):
                    rowsv = blk * LANES + iota16
                    acc = plsc.load_gather(
                        scr, [rowsv, jnp.zeros((LANES,), jnp.int32)])
                    for e in range(1, LANES):
                        acc = acc + plsc.load_gather(
                            scr, [rowsv, jnp.full((LANES,), e, jnp.int32)])
                    out_v[pl.ds(base_r + blk * LANES, LANES)] = acc

            if p + NSLOT < NP:
                pend[p + NSLOT] = fire(p + NSLOT)

        for c in range(C):
            @pl.loop(0, BPW // LANES)
            def _(j, c=c):
                bv = j * LANES + iota16_
                vals = plsc.load_gather(out_v, [bv * C + c])
                out_t[c, pl.ds(j * LANES, LANES)] = vals

        pltpu.sync_copy(out_t, out_hbm.at[:, pl.ds(b0, BPW)])

    return k(target, ctx_flat, ttab_packed, ctab_packed)


def kernel(target, context, target_table, context_table):
    B, = target.shape
    _, C = context.shape
    _, E = target_table.shape
    out_t = _anime2vec_sc(
        target,
        context.T,
        target_table.T,
        context_table.T,
        B=B, C=C, E=E)
    return out_t.T
